# baseline (speedup 1.0000x reference)
import functools

import jax
import jax.numpy as jnp
import numpy as np
from jax import lax
from jax.experimental import pallas as pl
from jax.experimental.pallas import tpu as pltpu
from jax.experimental.pallas import tpu_sc as plsc

N = 10000
E = 320000
D = 128
NC = 2
NS = 16
NW = NC * NS
NP = 10240
EPW = E // NW
EPT = E // NS
CH = 48
XW = D // 2
NCHA = EPW // CH
TAIL = EPW - NCHA * CH
SL = NP // NS
NA = 10112
SLA = NA // NS
RB = 256
DFD = 10

_mesh = plsc.VectorSubcoreMesh(core_axis_name="c", subcore_axis_name="s")
_sc_params = pltpu.CompilerParams(needs_layout_passes=False)
_sc_params_agg = pltpu.CompilerParams(
    needs_layout_passes=False, use_tc_tiling_on_sc=False)


def _zero_vec16():
    return jnp.zeros((16,), jnp.float32)


@functools.partial(
    pl.kernel,
    out_type=jax.ShapeDtypeStruct((E,), jnp.float32),
    mesh=_mesh,
    scratch_types=[
        pltpu.VMEM((EPW,), jnp.int32),
        pltpu.VMEM((EPT,), jnp.int32),
        pltpu.VMEM((EPT,), jnp.float32),
        pltpu.VMEM((EPW,), jnp.float32),
        pltpu.VMEM((NP,), jnp.float32),
        pltpu.VMEM((SL,), jnp.float32),
        pltpu.VMEM_SHARED((NP,), jnp.float32),
        pltpu.VMEM_SHARED((NP,), jnp.float32),
        pltpu.SemaphoreType.DMA,
        pltpu.SemaphoreType.DMA,
    ],
    compiler_params=_sc_params,
)
def _norm_kernel(src_hbm, dst_hbm, w_hbm, norm_hbm,
                 src_v, dst_v, w_v, nrm_v, dsq_v, sl_v, deg_sh, dsq_sh,
                 sem0, sem1):
    cid = lax.axis_index("c")
    sid = lax.axis_index("s")
    wid = sid * NC + cid

    c_s = pltpu.async_copy(src_hbm.at[pl.ds(wid * EPW, EPW)], src_v, sem0)
    c_d = pltpu.async_copy(dst_hbm.at[pl.ds(sid * EPT, EPT)], dst_v, sem0)
    c_w = pltpu.async_copy(w_hbm.at[pl.ds(sid * EPT, EPT)], w_v, sem0)

    def _z(k, _):
        sl_v[pl.ds(k * 16, 16)] = _zero_vec16()
        return 0
    lax.fori_loop(0, SL // 16, _z, 0)
    pltpu.sync_copy(sl_v, deg_sh.at[pl.ds(sid * SL, SL)])
    c_s.wait()
    c_d.wait()
    c_w.wait()
    plsc.subcore_barrier()

    def _dbatch(t, _):
        descs = []
        for j in range(DFD):
            k = (t * DFD + j) * 16
            idx16 = dst_v[pl.ds(k, 16)]
            descs.append(pltpu.async_copy(
                w_v.at[pl.ds(k, 16)], deg_sh.at[idx16], sem1, add=True))
        for d in descs:
            d.wait()
        return 0
    lax.fori_loop(0, EPT // 16 // DFD, _dbatch, 0)
    plsc.subcore_barrier()

    pltpu.sync_copy(deg_sh.at[pl.ds(sid * SL, SL)], sl_v)

    def _rs(k, _):
        x = jnp.maximum(sl_v[pl.ds(k * 16, 16)], 1e-12)
        i = lax.bitcast_convert_type(x, jnp.int32)
        i = 0x5F3759DF - lax.shift_right_logical(i, 1)
        y = lax.bitcast_convert_type(i, jnp.float32)
        for _ in range(3):
            y = y * (1.5 - 0.5 * x * y * y)
        sl_v[pl.ds(k * 16, 16)] = y
        return 0
    lax.fori_loop(0, SL // 16, _rs, 0)
    pltpu.sync_copy(sl_v, dsq_sh.at[pl.ds(sid * SL, SL)])
    plsc.subcore_barrier()

    pltpu.sync_copy(dsq_sh, dsq_v)
    off = cid * EPW

    def _ngrp(t, _):
        k = t * 16
        s16 = src_v[pl.ds(k, 16)]
        d16 = dst_v[pl.ds(off + k, 16)]
        ww = w_v[pl.ds(off + k, 16)]
        a = plsc.load_gather(dsq_v, [s16])
        b = plsc.load_gather(dsq_v, [d16])
        nrm_v[pl.ds(k, 16)] = ww * a * b
        return 0
    lax.fori_loop(0, EPW // 16, _ngrp, 0)
    pltpu.sync_copy(nrm_v, norm_hbm.at[pl.ds(wid * EPW, EPW)])


@functools.partial(
    pl.kernel,
    out_type=jax.ShapeDtypeStruct((NC, NA, D), jnp.float32),
    mesh=_mesh,
    scratch_types=[
        pltpu.VMEM((EPW,), jnp.int32),
        pltpu.VMEM((EPW,), jnp.int32),
        pltpu.VMEM((EPW,), jnp.float32),
        pltpu.VMEM((CH, XW), jnp.int32),
        pltpu.VMEM((CH, XW), jnp.int32),
        pltpu.VMEM((CH, D), jnp.float32),
        pltpu.VMEM((CH, D), jnp.float32),
        pltpu.VMEM((CH,), jnp.int32),
        pltpu.VMEM((CH,), jnp.int32),
        pltpu.VMEM_SHARED((NA, D), jnp.float32),
        pltpu.SemaphoreType.DMA,
        pltpu.SemaphoreType.DMA,
        pltpu.SemaphoreType.DMA,
        pltpu.SemaphoreType.DMA,
        pltpu.SemaphoreType.DMA,
        pltpu.SemaphoreType.DMA,
    ],
    compiler_params=_sc_params_agg,
)
def _agg_kernel(xs_hbm, src_hbm, dst_hbm, nrm_hbm, out_hbm,
                src_v, dst_v, nrm_v, rows_a, rows_b, frows_a, frows_b,
                dst_ca, dst_cb, agg_sh,
                sem0, sem_a, sem_b, sem_s, sem_ca, sem_cb):
    cid = lax.axis_index("c")
    sid = lax.axis_index("s")
    wid = sid * NC + cid

    c_s = pltpu.async_copy(src_hbm.at[pl.ds(wid * EPW, EPW)], src_v, sem0)
    c_d = pltpu.async_copy(dst_hbm.at[pl.ds(wid * EPW, EPW)], dst_v, sem0)
    c_n = pltpu.async_copy(nrm_hbm.at[pl.ds(wid * EPW, EPW)], nrm_v, sem0)

    def _z(k, _):
        frows_a[k // 8, pl.ds((k % 8) * 16, 16)] = _zero_vec16()
        return 0
    lax.fori_loop(0, CH * D // 16, _z, 0)
    for k in range(SLA // CH):
        pltpu.sync_copy(frows_a, agg_sh.at[pl.ds(sid * SLA + k * CH, CH)])
    rem = SLA % CH
    pltpu.sync_copy(frows_a.at[pl.ds(0, rem)],
                    agg_sh.at[pl.ds(sid * SLA + SLA - rem, rem)])
    c_s.wait()
    c_d.wait()
    c_n.wait()
    plsc.subcore_barrier()

    def _unpack_row(rows, i, spl, frows):
        for b in range(XW // 16):
            v = rows[i, pl.ds(b * 16, 16)]
            lo = lax.bitcast_convert_type(jnp.left_shift(v, 16), jnp.float32)
            hi = lax.bitcast_convert_type(
                jnp.bitwise_and(v, jnp.int32(-65536)), jnp.float32)
            frows[i, pl.ds(b * 32, 16)] = lo * spl
            frows[i, pl.ds(b * 32 + 16, 16)] = hi * spl

    def _scale(t, rows, frows):
        def _grp(g, _):
            k = t * CH + g * 16
            w16 = nrm_v[pl.ds(k, 16)]
            base = g * 16
            for r in range(16):
                spl = w16.at[lax.broadcast(r, (16,))].get(
                    mode="promise_in_bounds")
                _unpack_row(rows, base + r, spl, frows)
            return 0
        lax.fori_loop(0, CH // 16, _grp, 0)

    def _scatter(frows, dst_c):
        pltpu.async_copy(frows, agg_sh.at[dst_c], sem_s, add=True)
        pltpu.make_async_copy(frows, agg_sh.at[dst_c], sem_s).wait()

    def _gather(t, rows, dst_c, sem, sem_c):
        pltpu.async_copy(
            dst_hbm.at[pl.ds(wid * EPW + t * CH, CH)], dst_c, sem_c)
        return pltpu.async_copy(
            xs_hbm.at[src_v.at[pl.ds(t * CH, CH)]], rows, sem)

    def _wait_gather(t, rows, dst_c, sem, sem_c):
        pltpu.make_async_copy(
            dst_hbm.at[pl.ds(wid * EPW + t * CH, CH)], dst_c, sem_c).wait()
        pltpu.make_async_copy(
            xs_hbm.at[src_v.at[pl.ds(t * CH, CH)]], rows, sem).wait()

    _gather(0, rows_a, dst_ca, sem_a, sem_ca)

    def _pair(p, _):
        ta = 2 * p
        tb = 2 * p + 1
        _gather(tb, rows_b, dst_cb, sem_b, sem_cb)

        _wait_gather(ta, rows_a, dst_ca, sem_a, sem_ca)
        _scale(ta, rows_a, frows_a)
        _scatter(frows_a, dst_ca)

        @pl.when(p < NCHA // 2 - 1)
        def _():
            _gather(ta + 2, rows_a, dst_ca, sem_a, sem_ca)

        @pl.when(p == NCHA // 2 - 1)
        def _():
            pltpu.async_copy(
                xs_hbm.at[src_v.at[pl.ds(NCHA * CH, TAIL)]],
                rows_a.at[pl.ds(0, TAIL)], sem_a)

        _wait_gather(tb, rows_b, dst_cb, sem_b, sem_cb)
        _scale(tb, rows_b, frows_b)
        _scatter(frows_b, dst_cb)
        return 0
    lax.fori_loop(0, NCHA // 2, _pair, 0)

    tk = NCHA * CH
    pltpu.make_async_copy(
        xs_hbm.at[src_v.at[pl.ds(tk, TAIL)]],
        rows_a.at[pl.ds(0, TAIL)], sem_a).wait()
    w16 = nrm_v[pl.ds(tk, 16)]
    for r in range(16):
        spl = w16.at[lax.broadcast(r, (16,))].get(mode="promise_in_bounds")
        _unpack_row(rows_a, r, spl, frows_a)
    d16 = dst_v[pl.ds(tk, 16)]
    pltpu.async_copy(frows_a.at[pl.ds(0, TAIL)], agg_sh.at[d16],
                     sem_s, add=True).wait()

    plsc.subcore_barrier()
    pltpu.sync_copy(agg_sh.at[pl.ds(sid * SLA, SLA)],
                    out_hbm.at[cid, pl.ds(sid * SLA, SLA)])


def _mm_pre_body(x_ref, w_ref, b_ref, o_ref):
    acc = jnp.dot(x_ref[...], w_ref[...], preferred_element_type=jnp.float32)
    o_ref[...] = jnp.maximum(acc + b_ref[...], 0.0).astype(jnp.bfloat16)


def _mm_mid_body(p_ref, w_ref, b_ref, o_ref):
    s = p_ref[0] + p_ref[1]
    acc = jnp.dot(s, w_ref[...], preferred_element_type=jnp.float32)
    o_ref[...] = jnp.maximum(acc + b_ref[...], 0.0).astype(jnp.bfloat16)


def _mm_fin_body(p_ref, w_ref, b_ref, wp_ref, bp_ref, o_ref):
    s = p_ref[0] + p_ref[1]
    acc = jnp.dot(s, w_ref[...], preferred_element_type=jnp.float32)
    x = jnp.maximum(acc + b_ref[...], 0.0)
    o_ref[...] = jnp.dot(x, wp_ref[...],
                         preferred_element_type=jnp.float32) + bp_ref[...]


_w_spec = pl.BlockSpec((D, D), lambda i: (0, 0))
_b_spec = pl.BlockSpec((1, D), lambda i: (0, 0))
_row_spec = pl.BlockSpec((RB, D), lambda i: (i, 0))
_p_spec = pl.BlockSpec((NC, RB, D), lambda i: (0, i, 0))
_out_rows = jax.ShapeDtypeStruct((NP, D), jnp.float32)
_out_rows_bf = jax.ShapeDtypeStruct((NP, D), jnp.bfloat16)


def _tc_pre(x, w, b):
    return pl.pallas_call(
        _mm_pre_body, grid=(NP // RB,),
        in_specs=[_row_spec, _w_spec, _b_spec],
        out_specs=_row_spec, out_shape=_out_rows_bf,
    )(x, w, b)


def _tc_mid(p, w, b):
    return pl.pallas_call(
        _mm_mid_body, grid=(NP // RB,),
        in_specs=[_p_spec, _w_spec, _b_spec],
        out_specs=_row_spec, out_shape=_out_rows_bf,
    )(p, w, b)


def _tc_fin(p, w, b, wp, bp):
    return pl.pallas_call(
        _mm_fin_body, grid=(NP // RB,),
        in_specs=[_p_spec, _w_spec, _b_spec, _w_spec, _b_spec],
        out_specs=_row_spec, out_shape=_out_rows,
    )(p, w, b, wp, bp)


_PERM = np.concatenate([
    np.concatenate([np.arange(b * 32, b * 32 + 32, 2),
                    np.arange(b * 32 + 1, b * 32 + 32, 2)])
    for b in range(D // 32)
])


def _pack_rows(x):
    return lax.bitcast_convert_type(x.reshape(NP, XW, 2), jnp.int32)


def kernel(h, edge_index, edge_weight, W_pre, b_pre, W1, b1, W2, b2,
           W_post, b_post):
    src = edge_index[0].astype(jnp.int32)
    dst = edge_index[1].astype(jnp.int32)
    w = edge_weight.astype(jnp.float32)

    norm = _norm_kernel(src, dst, w)

    h_pad = jnp.pad(h, ((0, NP - N), (0, 0)))
    b_pre2 = b_pre.reshape(1, D)
    b12 = b1.reshape(1, D)
    b22 = b2.reshape(1, D)
    b_post2 = b_post.reshape(1, D)
    perm = jnp.asarray(_PERM)
    W1p = W1[perm, :]
    W2p = W2[perm, :]

    x1 = _tc_pre(h_pad, W_pre, b_pre2)
    p1 = _agg_kernel(_pack_rows(x1), src, dst, norm)
    x2 = _tc_mid(p1, W1p, b12)
    p2 = _agg_kernel(_pack_rows(x2), src, dst, norm)
    out = _tc_fin(p2, W2p, b22, W_post, b_post2)
    return out[:N]


# speedup vs baseline: 1.8750x; 1.8750x over previous
import functools

import jax
import jax.numpy as jnp
from jax import lax
from jax.experimental import pallas as pl
from jax.experimental.pallas import tpu as pltpu
from jax.experimental.pallas import tpu_sc as plsc

N = 10000
E = 320000
D = 128
NC = 2
NS = 16
NW = NC * NS
NP = 10240
EPW = E // NW
EPT = E // NS
CH = 64
NCHA = EPW // CH
TAIL = EPW - NCHA * CH
SL = NP // NS
NA = 10112
SLA = NA // NS
RB = 256
DFD = 10

_mesh = plsc.VectorSubcoreMesh(core_axis_name="c", subcore_axis_name="s")
_sc_params = pltpu.CompilerParams(needs_layout_passes=False)


def _zero_vec16():
    return jnp.zeros((16,), jnp.float32)


@functools.partial(
    pl.kernel,
    out_type=jax.ShapeDtypeStruct((E,), jnp.float32),
    mesh=_mesh,
    scratch_types=[
        pltpu.VMEM((EPW,), jnp.int32),
        pltpu.VMEM((EPT,), jnp.int32),
        pltpu.VMEM((EPT,), jnp.float32),
        pltpu.VMEM((EPW,), jnp.float32),
        pltpu.VMEM((NP,), jnp.float32),
        pltpu.VMEM((SL,), jnp.float32),
        pltpu.VMEM_SHARED((NP,), jnp.float32),
        pltpu.VMEM_SHARED((NP,), jnp.float32),
        pltpu.SemaphoreType.DMA,
        pltpu.SemaphoreType.DMA,
    ],
    compiler_params=_sc_params,
)
def _norm_kernel(src_hbm, dst_hbm, w_hbm, norm_hbm,
                 src_v, dst_v, w_v, nrm_v, dsq_v, sl_v, deg_sh, dsq_sh,
                 sem0, sem1):
    cid = lax.axis_index("c")
    sid = lax.axis_index("s")
    wid = sid * NC + cid

    c_s = pltpu.async_copy(src_hbm.at[pl.ds(wid * EPW, EPW)], src_v, sem0)
    c_d = pltpu.async_copy(dst_hbm.at[pl.ds(sid * EPT, EPT)], dst_v, sem0)
    c_w = pltpu.async_copy(w_hbm.at[pl.ds(sid * EPT, EPT)], w_v, sem0)

    def _z(k, _):
        sl_v[pl.ds(k * 16, 16)] = _zero_vec16()
        return 0
    lax.fori_loop(0, SL // 16, _z, 0)
    pltpu.sync_copy(sl_v, deg_sh.at[pl.ds(sid * SL, SL)])
    c_s.wait()
    c_d.wait()
    c_w.wait()
    plsc.subcore_barrier()

    def _dbatch(t, _):
        descs = []
        for j in range(DFD):
            k = (t * DFD + j) * 16
            idx16 = dst_v[pl.ds(k, 16)]
            descs.append(pltpu.async_copy(
                w_v.at[pl.ds(k, 16)], deg_sh.at[idx16], sem1, add=True))
        for d in descs:
            d.wait()
        return 0
    lax.fori_loop(0, EPT // 16 // DFD, _dbatch, 0)
    plsc.subcore_barrier()

    pltpu.sync_copy(deg_sh.at[pl.ds(sid * SL, SL)], sl_v)

    def _rs(k, _):
        x = jnp.maximum(sl_v[pl.ds(k * 16, 16)], 1e-12)
        i = lax.bitcast_convert_type(x, jnp.int32)
        i = 0x5F3759DF - lax.shift_right_logical(i, 1)
        y = lax.bitcast_convert_type(i, jnp.float32)
        for _ in range(3):
            y = y * (1.5 - 0.5 * x * y * y)
        sl_v[pl.ds(k * 16, 16)] = y
        return 0
    lax.fori_loop(0, SL // 16, _rs, 0)
    pltpu.sync_copy(sl_v, dsq_sh.at[pl.ds(sid * SL, SL)])
    plsc.subcore_barrier()

    pltpu.sync_copy(dsq_sh, dsq_v)
    off = cid * EPW

    def _ngrp(t, _):
        k = t * 16
        s16 = src_v[pl.ds(k, 16)]
        d16 = dst_v[pl.ds(off + k, 16)]
        ww = w_v[pl.ds(off + k, 16)]
        a = plsc.load_gather(dsq_v, [s16])
        b = plsc.load_gather(dsq_v, [d16])
        nrm_v[pl.ds(k, 16)] = ww * a * b
        return 0
    lax.fori_loop(0, EPW // 16, _ngrp, 0)
    pltpu.sync_copy(nrm_v, norm_hbm.at[pl.ds(wid * EPW, EPW)])


@functools.partial(
    pl.kernel,
    out_type=jax.ShapeDtypeStruct((NC, NA, D), jnp.float32),
    mesh=_mesh,
    scratch_types=[
        pltpu.VMEM((EPW,), jnp.int32),
        pltpu.VMEM((EPW,), jnp.int32),
        pltpu.VMEM((EPW,), jnp.float32),
        pltpu.VMEM((CH, D), jnp.float32),
        pltpu.VMEM((CH, D), jnp.float32),
        pltpu.VMEM((CH,), jnp.int32),
        pltpu.VMEM((CH,), jnp.int32),
        pltpu.VMEM_SHARED((NA, D), jnp.float32),
        pltpu.SemaphoreType.DMA,
        pltpu.SemaphoreType.DMA,
        pltpu.SemaphoreType.DMA,
        pltpu.SemaphoreType.DMA,
        pltpu.SemaphoreType.DMA,
        pltpu.SemaphoreType.DMA,
    ],
    compiler_params=_sc_params,
)
def _agg_kernel(xs_hbm, src_hbm, dst_hbm, nrm_hbm, out_hbm,
                src_v, dst_v, nrm_v, rows_a, rows_b, dst_ca, dst_cb, agg_sh,
                sem0, sem_a, sem_b, sem_s, sem_ca, sem_cb):
    cid = lax.axis_index("c")
    sid = lax.axis_index("s")
    wid = sid * NC + cid

    c_s = pltpu.async_copy(src_hbm.at[pl.ds(wid * EPW, EPW)], src_v, sem0)
    c_d = pltpu.async_copy(dst_hbm.at[pl.ds(wid * EPW, EPW)], dst_v, sem0)
    c_n = pltpu.async_copy(nrm_hbm.at[pl.ds(wid * EPW, EPW)], nrm_v, sem0)

    def _z(k, _):
        rows_a[k // 8, pl.ds((k % 8) * 16, 16)] = _zero_vec16()
        return 0
    lax.fori_loop(0, CH * D // 16, _z, 0)
    for k in range(SLA // CH):
        pltpu.sync_copy(rows_a, agg_sh.at[pl.ds(sid * SLA + k * CH, CH)])
    rem = SLA % CH
    pltpu.sync_copy(rows_a.at[pl.ds(0, rem)],
                    agg_sh.at[pl.ds(sid * SLA + SLA - rem, rem)])
    c_s.wait()
    c_d.wait()
    c_n.wait()
    plsc.subcore_barrier()

    def _scale(t, rows):
        def _grp(g, _):
            k = t * CH + g * 16
            w16 = nrm_v[pl.ds(k, 16)]
            base = g * 16
            for r in range(16):
                spl = w16.at[lax.broadcast(r, (16,))].get(
                    mode="promise_in_bounds")
                for f in range(D // 16):
                    rows[base + r, pl.ds(f * 16, 16)] = (
                        rows[base + r, pl.ds(f * 16, 16)] * spl)
            return 0
        lax.fori_loop(0, CH // 16, _grp, 0)

    def _scatter(rows, dst_c):
        pltpu.async_copy(rows, agg_sh.at[dst_c], sem_s, add=True)
        pltpu.make_async_copy(rows, agg_sh.at[dst_c], sem_s).wait()

    HCH = CH // 2

    def _gather(t, rows, dst_c, sem, sem_c):
        pltpu.async_copy(
            dst_hbm.at[pl.ds(wid * EPW + t * CH, CH)], dst_c, sem_c)
        pltpu.async_copy(
            xs_hbm.at[src_v.at[pl.ds(t * CH, HCH)]],
            rows.at[pl.ds(0, HCH)], sem)
        pltpu.async_copy(
            xs_hbm.at[src_v.at[pl.ds(t * CH + HCH, HCH)]],
            rows.at[pl.ds(HCH, HCH)], sem)

    def _wait_gather(t, rows, dst_c, sem, sem_c):
        pltpu.make_async_copy(
            dst_hbm.at[pl.ds(wid * EPW + t * CH, CH)], dst_c, sem_c).wait()
        pltpu.make_async_copy(
            xs_hbm.at[src_v.at[pl.ds(t * CH, HCH)]],
            rows.at[pl.ds(0, HCH)], sem).wait()
        pltpu.make_async_copy(
            xs_hbm.at[src_v.at[pl.ds(t * CH + HCH, HCH)]],
            rows.at[pl.ds(HCH, HCH)], sem).wait()

    _gather(0, rows_a, dst_ca, sem_a, sem_ca)

    def _pair(p, _):
        ta = 2 * p
        tb = 2 * p + 1
        _gather(tb, rows_b, dst_cb, sem_b, sem_cb)

        _wait_gather(ta, rows_a, dst_ca, sem_a, sem_ca)
        _scale(ta, rows_a)
        _scatter(rows_a, dst_ca)

        @pl.when(p < NCHA // 2 - 1)
        def _():
            _gather(ta + 2, rows_a, dst_ca, sem_a, sem_ca)

        @pl.when(p == NCHA // 2 - 1)
        def _():
            pltpu.async_copy(
                xs_hbm.at[src_v.at[pl.ds(NCHA * CH, TAIL)]],
                rows_a.at[pl.ds(0, TAIL)], sem_a)

        _wait_gather(tb, rows_b, dst_cb, sem_b, sem_cb)
        _scale(tb, rows_b)
        _scatter(rows_b, dst_cb)
        return 0
    lax.fori_loop(0, NCHA // 2, _pair, 0)

    tk = NCHA * CH
    pltpu.make_async_copy(
        xs_hbm.at[src_v.at[pl.ds(tk, TAIL)]],
        rows_a.at[pl.ds(0, TAIL)], sem_a).wait()
    w16 = nrm_v[pl.ds(tk, 16)]
    for r in range(16):
        spl = w16.at[lax.broadcast(r, (16,))].get(mode="promise_in_bounds")
        for f in range(D // 16):
            rows_a[r, pl.ds(f * 16, 16)] = rows_a[r, pl.ds(f * 16, 16)] * spl
    d16 = dst_v[pl.ds(tk, 16)]
    pltpu.async_copy(rows_a.at[pl.ds(0, TAIL)], agg_sh.at[d16],
                     sem_s, add=True).wait()

    plsc.subcore_barrier()
    pltpu.sync_copy(agg_sh.at[pl.ds(sid * SLA, SLA)],
                    out_hbm.at[cid, pl.ds(sid * SLA, SLA)])


def _mm_pre_body(x_ref, w_ref, b_ref, o_ref):
    acc = jnp.dot(x_ref[...], w_ref[...], preferred_element_type=jnp.float32)
    o_ref[...] = jnp.maximum(acc + b_ref[...], 0.0)


def _mm_mid_body(p_ref, w_ref, b_ref, o_ref):
    s = p_ref[0] + p_ref[1]
    acc = jnp.dot(s, w_ref[...], preferred_element_type=jnp.float32)
    o_ref[...] = jnp.maximum(acc + b_ref[...], 0.0)


def _mm_fin_body(p_ref, w_ref, b_ref, wp_ref, bp_ref, o_ref):
    s = p_ref[0] + p_ref[1]
    acc = jnp.dot(s, w_ref[...], preferred_element_type=jnp.float32)
    x = jnp.maximum(acc + b_ref[...], 0.0)
    o_ref[...] = jnp.dot(x, wp_ref[...],
                         preferred_element_type=jnp.float32) + bp_ref[...]


_w_spec = pl.BlockSpec((D, D), lambda i: (0, 0))
_b_spec = pl.BlockSpec((1, D), lambda i: (0, 0))
_row_spec = pl.BlockSpec((RB, D), lambda i: (i, 0))
_p_spec = pl.BlockSpec((NC, RB, D), lambda i: (0, i, 0))
_out_rows = jax.ShapeDtypeStruct((NP, D), jnp.float32)


def _tc_pre(x, w, b):
    return pl.pallas_call(
        _mm_pre_body, grid=(NP // RB,),
        in_specs=[_row_spec, _w_spec, _b_spec],
        out_specs=_row_spec, out_shape=_out_rows,
    )(x, w, b)


def _tc_mid(p, w, b):
    return pl.pallas_call(
        _mm_mid_body, grid=(NP // RB,),
        in_specs=[_p_spec, _w_spec, _b_spec],
        out_specs=_row_spec, out_shape=_out_rows,
    )(p, w, b)


def _tc_fin(p, w, b, wp, bp):
    return pl.pallas_call(
        _mm_fin_body, grid=(NP // RB,),
        in_specs=[_p_spec, _w_spec, _b_spec, _w_spec, _b_spec],
        out_specs=_row_spec, out_shape=_out_rows,
    )(p, w, b, wp, bp)


def kernel(h, edge_index, edge_weight, W_pre, b_pre, W1, b1, W2, b2,
           W_post, b_post):
    src = edge_index[0].astype(jnp.int32)
    dst = edge_index[1].astype(jnp.int32)
    w = edge_weight.astype(jnp.float32)

    norm = _norm_kernel(src, dst, w)

    h_pad = jnp.pad(h, ((0, NP - N), (0, 0)))
    b_pre2 = b_pre.reshape(1, D)
    b12 = b1.reshape(1, D)
    b22 = b2.reshape(1, D)
    b_post2 = b_post.reshape(1, D)

    x1 = _tc_pre(h_pad, W_pre, b_pre2)
    p1 = _agg_kernel(x1, src, dst, norm)
    x2 = _tc_mid(p1, W1, b12)
    p2 = _agg_kernel(x2, src, dst, norm)
    out = _tc_fin(p2, W2, b22, W_post, b_post2)
    return out[:N]


# speedup vs baseline: 2.0202x; 1.0774x over previous
import functools

import jax
import jax.numpy as jnp
from jax import lax
from jax.experimental import pallas as pl
from jax.experimental.pallas import tpu as pltpu
from jax.experimental.pallas import tpu_sc as plsc

N = 10000
E = 320000
D = 128
NC = 2
NS = 16
NW = NC * NS
NP = 10240
EPW = E // NW
EPT = E // NS
CH = 96
NCHA = EPW // CH
TAIL = EPW - NCHA * CH
SL = NP // NS
NA = 10112
SLA = NA // NS
RB = 256
DFD = 10

_mesh = plsc.VectorSubcoreMesh(core_axis_name="c", subcore_axis_name="s")
_sc_params = pltpu.CompilerParams(needs_layout_passes=False)


def _zero_vec16():
    return jnp.zeros((16,), jnp.float32)


@functools.partial(
    pl.kernel,
    out_type=jax.ShapeDtypeStruct((E,), jnp.float32),
    mesh=_mesh,
    scratch_types=[
        pltpu.VMEM((EPW,), jnp.int32),
        pltpu.VMEM((EPT,), jnp.int32),
        pltpu.VMEM((EPT,), jnp.float32),
        pltpu.VMEM((EPW,), jnp.float32),
        pltpu.VMEM((NP,), jnp.float32),
        pltpu.VMEM((SL,), jnp.float32),
        pltpu.VMEM_SHARED((NP,), jnp.float32),
        pltpu.VMEM_SHARED((NP,), jnp.float32),
        pltpu.SemaphoreType.DMA,
        pltpu.SemaphoreType.DMA,
    ],
    compiler_params=_sc_params,
)
def _norm_kernel(src_hbm, dst_hbm, w_hbm, norm_hbm,
                 src_v, dst_v, w_v, nrm_v, dsq_v, sl_v, deg_sh, dsq_sh,
                 sem0, sem1):
    cid = lax.axis_index("c")
    sid = lax.axis_index("s")
    wid = sid * NC + cid

    c_s = pltpu.async_copy(src_hbm.at[pl.ds(wid * EPW, EPW)], src_v, sem0)
    c_d = pltpu.async_copy(dst_hbm.at[pl.ds(sid * EPT, EPT)], dst_v, sem0)
    c_w = pltpu.async_copy(w_hbm.at[pl.ds(sid * EPT, EPT)], w_v, sem0)

    def _z(k, _):
        sl_v[pl.ds(k * 16, 16)] = _zero_vec16()
        return 0
    lax.fori_loop(0, SL // 16, _z, 0)
    pltpu.sync_copy(sl_v, deg_sh.at[pl.ds(sid * SL, SL)])
    c_s.wait()
    c_d.wait()
    c_w.wait()
    plsc.subcore_barrier()

    def _dbatch(t, _):
        descs = []
        for j in range(DFD):
            k = (t * DFD + j) * 16
            idx16 = dst_v[pl.ds(k, 16)]
            descs.append(pltpu.async_copy(
                w_v.at[pl.ds(k, 16)], deg_sh.at[idx16], sem1, add=True))
        for d in descs:
            d.wait()
        return 0
    lax.fori_loop(0, EPT // 16 // DFD, _dbatch, 0)
    plsc.subcore_barrier()

    pltpu.sync_copy(deg_sh.at[pl.ds(sid * SL, SL)], sl_v)

    def _rs(k, _):
        x = jnp.maximum(sl_v[pl.ds(k * 16, 16)], 1e-12)
        i = lax.bitcast_convert_type(x, jnp.int32)
        i = 0x5F3759DF - lax.shift_right_logical(i, 1)
        y = lax.bitcast_convert_type(i, jnp.float32)
        for _ in range(3):
            y = y * (1.5 - 0.5 * x * y * y)
        sl_v[pl.ds(k * 16, 16)] = y
        return 0
    lax.fori_loop(0, SL // 16, _rs, 0)
    pltpu.sync_copy(sl_v, dsq_sh.at[pl.ds(sid * SL, SL)])
    plsc.subcore_barrier()

    pltpu.sync_copy(dsq_sh, dsq_v)
    off = cid * EPW

    def _ngrp(t, _):
        k = t * 16
        s16 = src_v[pl.ds(k, 16)]
        d16 = dst_v[pl.ds(off + k, 16)]
        ww = w_v[pl.ds(off + k, 16)]
        a = plsc.load_gather(dsq_v, [s16])
        b = plsc.load_gather(dsq_v, [d16])
        nrm_v[pl.ds(k, 16)] = ww * a * b
        return 0
    lax.fori_loop(0, EPW // 16, _ngrp, 0)
    pltpu.sync_copy(nrm_v, norm_hbm.at[pl.ds(wid * EPW, EPW)])


@functools.partial(
    pl.kernel,
    out_type=jax.ShapeDtypeStruct((NC, NA, D), jnp.float32),
    mesh=_mesh,
    scratch_types=[
        pltpu.VMEM((EPW,), jnp.int32),
        pltpu.VMEM((16,), jnp.int32),
        pltpu.VMEM((EPW,), jnp.float32),
        pltpu.VMEM((CH, D), jnp.float32),
        pltpu.VMEM((CH, D), jnp.float32),
        pltpu.VMEM((CH,), jnp.int32),
        pltpu.VMEM((CH,), jnp.int32),
        pltpu.VMEM_SHARED((NA, D), jnp.float32),
        pltpu.SemaphoreType.DMA,
        pltpu.SemaphoreType.DMA,
        pltpu.SemaphoreType.DMA,
        pltpu.SemaphoreType.DMA,
        pltpu.SemaphoreType.DMA,
        pltpu.SemaphoreType.DMA,
    ],
    compiler_params=_sc_params,
)
def _agg_kernel(xs_hbm, src_hbm, dst_hbm, nrm_hbm, out_hbm,
                src_v, dst_t, nrm_v, rows_a, rows_b, dst_ca, dst_cb, agg_sh,
                sem0, sem_a, sem_b, sem_s, sem_ca, sem_cb):
    cid = lax.axis_index("c")
    sid = lax.axis_index("s")
    wid = sid * NC + cid

    c_s = pltpu.async_copy(src_hbm.at[pl.ds(wid * EPW, EPW)], src_v, sem0)
    c_d = pltpu.async_copy(
        dst_hbm.at[pl.ds(wid * EPW + NCHA * CH, TAIL)], dst_t, sem0)
    c_n = pltpu.async_copy(nrm_hbm.at[pl.ds(wid * EPW, EPW)], nrm_v, sem0)

    def _z(k, _):
        rows_a[k // 8, pl.ds((k % 8) * 16, 16)] = _zero_vec16()
        return 0
    lax.fori_loop(0, CH * D // 16, _z, 0)
    for k in range(SLA // CH):
        pltpu.sync_copy(rows_a, agg_sh.at[pl.ds(sid * SLA + k * CH, CH)])
    rem = SLA % CH
    pltpu.sync_copy(rows_a.at[pl.ds(0, rem)],
                    agg_sh.at[pl.ds(sid * SLA + SLA - rem, rem)])
    c_s.wait()
    c_d.wait()
    c_n.wait()
    plsc.subcore_barrier()

    def _scale(t, rows):
        def _grp(g, _):
            k = t * CH + g * 16
            w16 = nrm_v[pl.ds(k, 16)]
            base = g * 16
            for r in range(16):
                spl = w16.at[lax.broadcast(r, (16,))].get(
                    mode="promise_in_bounds")
                for f in range(D // 16):
                    rows[base + r, pl.ds(f * 16, 16)] = (
                        rows[base + r, pl.ds(f * 16, 16)] * spl)
            return 0
        lax.fori_loop(0, CH // 16, _grp, 0)

    def _scatter(rows, dst_c):
        pltpu.async_copy(rows, agg_sh.at[dst_c], sem_s, add=True)
        pltpu.make_async_copy(rows, agg_sh.at[dst_c], sem_s).wait()

    HCH = CH // 2

    def _gather(t, rows, dst_c, sem, sem_c):
        pltpu.async_copy(
            dst_hbm.at[pl.ds(wid * EPW + t * CH, CH)], dst_c, sem_c)
        pltpu.async_copy(
            xs_hbm.at[src_v.at[pl.ds(t * CH, HCH)]],
            rows.at[pl.ds(0, HCH)], sem)
        pltpu.async_copy(
            xs_hbm.at[src_v.at[pl.ds(t * CH + HCH, HCH)]],
            rows.at[pl.ds(HCH, HCH)], sem)

    def _wait_gather(t, rows, dst_c, sem, sem_c):
        pltpu.make_async_copy(
            dst_hbm.at[pl.ds(wid * EPW + t * CH, CH)], dst_c, sem_c).wait()
        pltpu.make_async_copy(
            xs_hbm.at[src_v.at[pl.ds(t * CH, HCH)]],
            rows.at[pl.ds(0, HCH)], sem).wait()
        pltpu.make_async_copy(
            xs_hbm.at[src_v.at[pl.ds(t * CH + HCH, HCH)]],
            rows.at[pl.ds(HCH, HCH)], sem).wait()

    _gather(0, rows_a, dst_ca, sem_a, sem_ca)

    def _pair(p, _):
        ta = 2 * p
        tb = 2 * p + 1
        _gather(tb, rows_b, dst_cb, sem_b, sem_cb)

        _wait_gather(ta, rows_a, dst_ca, sem_a, sem_ca)
        _scale(ta, rows_a)
        _scatter(rows_a, dst_ca)

        @pl.when(p < NCHA // 2 - 1)
        def _():
            _gather(ta + 2, rows_a, dst_ca, sem_a, sem_ca)

        @pl.when(p == NCHA // 2 - 1)
        def _():
            pltpu.async_copy(
                xs_hbm.at[src_v.at[pl.ds(NCHA * CH, TAIL)]],
                rows_a.at[pl.ds(0, TAIL)], sem_a)

        _wait_gather(tb, rows_b, dst_cb, sem_b, sem_cb)
        _scale(tb, rows_b)
        _scatter(rows_b, dst_cb)
        return 0
    lax.fori_loop(0, NCHA // 2, _pair, 0)

    tk = NCHA * CH
    pltpu.make_async_copy(
        xs_hbm.at[src_v.at[pl.ds(tk, TAIL)]],
        rows_a.at[pl.ds(0, TAIL)], sem_a).wait()
    w16 = nrm_v[pl.ds(tk, 16)]
    for r in range(16):
        spl = w16.at[lax.broadcast(r, (16,))].get(mode="promise_in_bounds")
        for f in range(D // 16):
            rows_a[r, pl.ds(f * 16, 16)] = rows_a[r, pl.ds(f * 16, 16)] * spl
    pltpu.async_copy(rows_a.at[pl.ds(0, TAIL)], agg_sh.at[dst_t],
                     sem_s, add=True).wait()

    plsc.subcore_barrier()
    pltpu.sync_copy(agg_sh.at[pl.ds(sid * SLA, SLA)],
                    out_hbm.at[cid, pl.ds(sid * SLA, SLA)])


def _mm_pre_body(x_ref, w_ref, b_ref, o_ref):
    acc = jnp.dot(x_ref[...], w_ref[...], preferred_element_type=jnp.float32)
    o_ref[...] = jnp.maximum(acc + b_ref[...], 0.0)


def _mm_mid_body(p_ref, w_ref, b_ref, o_ref):
    s = p_ref[0] + p_ref[1]
    acc = jnp.dot(s, w_ref[...], preferred_element_type=jnp.float32)
    o_ref[...] = jnp.maximum(acc + b_ref[...], 0.0)


def _mm_fin_body(p_ref, w_ref, b_ref, wp_ref, bp_ref, o_ref):
    s = p_ref[0] + p_ref[1]
    acc = jnp.dot(s, w_ref[...], preferred_element_type=jnp.float32)
    x = jnp.maximum(acc + b_ref[...], 0.0)
    o_ref[...] = jnp.dot(x, wp_ref[...],
                         preferred_element_type=jnp.float32) + bp_ref[...]


_w_spec = pl.BlockSpec((D, D), lambda i: (0, 0))
_b_spec = pl.BlockSpec((1, D), lambda i: (0, 0))
_row_spec = pl.BlockSpec((RB, D), lambda i: (i, 0))
_p_spec = pl.BlockSpec((NC, RB, D), lambda i: (0, i, 0))
_out_rows = jax.ShapeDtypeStruct((NP, D), jnp.float32)


def _tc_pre(x, w, b):
    return pl.pallas_call(
        _mm_pre_body, grid=(NP // RB,),
        in_specs=[_row_spec, _w_spec, _b_spec],
        out_specs=_row_spec, out_shape=_out_rows,
    )(x, w, b)


def _tc_mid(p, w, b):
    return pl.pallas_call(
        _mm_mid_body, grid=(NP // RB,),
        in_specs=[_p_spec, _w_spec, _b_spec],
        out_specs=_row_spec, out_shape=_out_rows,
    )(p, w, b)


def _tc_fin(p, w, b, wp, bp):
    return pl.pallas_call(
        _mm_fin_body, grid=(NP // RB,),
        in_specs=[_p_spec, _w_spec, _b_spec, _w_spec, _b_spec],
        out_specs=_row_spec, out_shape=_out_rows,
    )(p, w, b, wp, bp)


def kernel(h, edge_index, edge_weight, W_pre, b_pre, W1, b1, W2, b2,
           W_post, b_post):
    src = edge_index[0].astype(jnp.int32)
    dst = edge_index[1].astype(jnp.int32)
    w = edge_weight.astype(jnp.float32)

    norm = _norm_kernel(src, dst, w)

    h_pad = jnp.pad(h, ((0, NP - N), (0, 0)))
    b_pre2 = b_pre.reshape(1, D)
    b12 = b1.reshape(1, D)
    b22 = b2.reshape(1, D)
    b_post2 = b_post.reshape(1, D)

    x1 = _tc_pre(h_pad, W_pre, b_pre2)
    p1 = _agg_kernel(x1, src, dst, norm)
    x2 = _tc_mid(p1, W1, b12)
    p2 = _agg_kernel(x2, src, dst, norm)
    out = _tc_fin(p2, W2, b22, W_post, b_post2)
    return out[:N]


# speedup vs baseline: 2.1612x; 1.0698x over previous
import functools

import jax
import jax.numpy as jnp
from jax import lax
from jax.experimental import pallas as pl
from jax.experimental.pallas import tpu as pltpu
from jax.experimental.pallas import tpu_sc as plsc

N = 10000
E = 320000
D = 128
NC = 2
NS = 16
NW = NC * NS
NP = 10240
EPW = E // NW
EPT = E // NS
CH = 96
NCHA = EPW // CH
TAIL = EPW - NCHA * CH
SL = NP // NS
NA = 10112
SLA = NA // NS
RB = 512
DFD = 20

_mesh = plsc.VectorSubcoreMesh(core_axis_name="c", subcore_axis_name="s")
_sc_params = pltpu.CompilerParams(needs_layout_passes=False)


def _zero_vec16():
    return jnp.zeros((16,), jnp.float32)


@functools.partial(
    pl.kernel,
    out_type=jax.ShapeDtypeStruct((E,), jnp.float32),
    mesh=_mesh,
    scratch_types=[
        pltpu.VMEM((EPW,), jnp.int32),
        pltpu.VMEM((EPT,), jnp.int32),
        pltpu.VMEM((EPT,), jnp.float32),
        pltpu.VMEM((EPW,), jnp.float32),
        pltpu.VMEM((NP,), jnp.float32),
        pltpu.VMEM((SL,), jnp.float32),
        pltpu.VMEM_SHARED((NP,), jnp.float32),
        pltpu.VMEM_SHARED((NP,), jnp.float32),
        pltpu.SemaphoreType.DMA,
        pltpu.SemaphoreType.DMA,
    ],
    compiler_params=_sc_params,
)
def _norm_kernel(src_hbm, dst_hbm, w_hbm, norm_hbm,
                 src_v, dst_v, w_v, nrm_v, dsq_v, sl_v, deg_sh, dsq_sh,
                 sem0, sem1):
    cid = lax.axis_index("c")
    sid = lax.axis_index("s")
    wid = sid * NC + cid

    c_s = pltpu.async_copy(src_hbm.at[pl.ds(wid * EPW, EPW)], src_v, sem0)
    c_d = pltpu.async_copy(dst_hbm.at[pl.ds(sid * EPT, EPT)], dst_v, sem0)
    c_w = pltpu.async_copy(w_hbm.at[pl.ds(sid * EPT, EPT)], w_v, sem0)

    def _z(k, _):
        sl_v[pl.ds(k * 16, 16)] = _zero_vec16()
        return 0
    lax.fori_loop(0, SL // 16, _z, 0)
    pltpu.sync_copy(sl_v, deg_sh.at[pl.ds(sid * SL, SL)])
    c_s.wait()
    c_d.wait()
    c_w.wait()
    plsc.subcore_barrier()

    def _dbatch(t, _):
        descs = []
        for j in range(DFD):
            k = (t * DFD + j) * 16
            idx16 = dst_v[pl.ds(k, 16)]
            descs.append(pltpu.async_copy(
                w_v.at[pl.ds(k, 16)], deg_sh.at[idx16], sem1, add=True))
        for d in descs:
            d.wait()
        return 0
    lax.fori_loop(0, EPT // 16 // DFD, _dbatch, 0)
    plsc.subcore_barrier()

    pltpu.sync_copy(deg_sh.at[pl.ds(sid * SL, SL)], sl_v)

    def _rs(k, _):
        x = jnp.maximum(sl_v[pl.ds(k * 16, 16)], 1e-12)
        i = lax.bitcast_convert_type(x, jnp.int32)
        i = 0x5F3759DF - lax.shift_right_logical(i, 1)
        y = lax.bitcast_convert_type(i, jnp.float32)
        for _ in range(3):
            y = y * (1.5 - 0.5 * x * y * y)
        sl_v[pl.ds(k * 16, 16)] = y
        return 0
    lax.fori_loop(0, SL // 16, _rs, 0)
    pltpu.sync_copy(sl_v, dsq_sh.at[pl.ds(sid * SL, SL)])
    plsc.subcore_barrier()

    pltpu.sync_copy(dsq_sh, dsq_v)
    off = cid * EPW

    def _ngrp(t, _):
        k = t * 16
        s16 = src_v[pl.ds(k, 16)]
        d16 = dst_v[pl.ds(off + k, 16)]
        ww = w_v[pl.ds(off + k, 16)]
        a = plsc.load_gather(dsq_v, [s16])
        b = plsc.load_gather(dsq_v, [d16])
        nrm_v[pl.ds(k, 16)] = ww * a * b
        return 0
    lax.fori_loop(0, EPW // 16, _ngrp, 0)
    pltpu.sync_copy(nrm_v, norm_hbm.at[pl.ds(wid * EPW, EPW)])


@functools.partial(
    pl.kernel,
    out_type=jax.ShapeDtypeStruct((NC, NA, D), jnp.float32),
    mesh=_mesh,
    scratch_types=[
        pltpu.VMEM((EPW,), jnp.int32),
        pltpu.VMEM((16,), jnp.int32),
        pltpu.VMEM((EPW,), jnp.float32),
        pltpu.VMEM((CH, D), jnp.float32),
        pltpu.VMEM((CH, D), jnp.float32),
        pltpu.VMEM((CH,), jnp.int32),
        pltpu.VMEM((CH,), jnp.int32),
        pltpu.VMEM_SHARED((NA, D), jnp.float32),
        pltpu.SemaphoreType.DMA,
        pltpu.SemaphoreType.DMA,
        pltpu.SemaphoreType.DMA,
        pltpu.SemaphoreType.DMA,
        pltpu.SemaphoreType.DMA,
        pltpu.SemaphoreType.DMA,
    ],
    compiler_params=_sc_params,
)
def _agg_kernel(xs_hbm, src_hbm, dst_hbm, nrm_hbm, out_hbm,
                src_v, dst_t, nrm_v, rows_a, rows_b, dst_ca, dst_cb, agg_sh,
                sem0, sem_a, sem_b, sem_s, sem_ca, sem_cb):
    cid = lax.axis_index("c")
    sid = lax.axis_index("s")
    wid = sid * NC + cid

    c_s = pltpu.async_copy(src_hbm.at[pl.ds(wid * EPW, EPW)], src_v, sem0)
    c_d = pltpu.async_copy(
        dst_hbm.at[pl.ds(wid * EPW + NCHA * CH, TAIL)], dst_t, sem0)
    c_n = pltpu.async_copy(nrm_hbm.at[pl.ds(wid * EPW, EPW)], nrm_v, sem0)

    def _z(k, _):
        rows_a[k // 8, pl.ds((k % 8) * 16, 16)] = _zero_vec16()
        return 0
    lax.fori_loop(0, CH * D // 16, _z, 0)
    for k in range(SLA // CH):
        pltpu.sync_copy(rows_a, agg_sh.at[pl.ds(sid * SLA + k * CH, CH)])
    rem = SLA % CH
    pltpu.sync_copy(rows_a.at[pl.ds(0, rem)],
                    agg_sh.at[pl.ds(sid * SLA + SLA - rem, rem)])
    c_s.wait()
    c_d.wait()
    c_n.wait()
    plsc.subcore_barrier()

    def _scale(t, rows):
        def _grp(g, _):
            k = t * CH + g * 16
            w16 = nrm_v[pl.ds(k, 16)]
            base = g * 16
            for r in range(16):
                spl = w16.at[lax.broadcast(r, (16,))].get(
                    mode="promise_in_bounds")
                for f in range(D // 16):
                    rows[base + r, pl.ds(f * 16, 16)] = (
                        rows[base + r, pl.ds(f * 16, 16)] * spl)
            return 0
        lax.fori_loop(0, CH // 16, _grp, 0)

    def _scatter(rows, dst_c):
        pltpu.async_copy(rows, agg_sh.at[dst_c], sem_s, add=True)
        pltpu.make_async_copy(rows, agg_sh.at[dst_c], sem_s).wait()

    HCH = CH // 2

    def _gather(t, rows, dst_c, sem, sem_c):
        pltpu.async_copy(
            dst_hbm.at[pl.ds(wid * EPW + t * CH, CH)], dst_c, sem_c)
        pltpu.async_copy(
            xs_hbm.at[src_v.at[pl.ds(t * CH, HCH)]],
            rows.at[pl.ds(0, HCH)], sem)
        pltpu.async_copy(
            xs_hbm.at[src_v.at[pl.ds(t * CH + HCH, HCH)]],
            rows.at[pl.ds(HCH, HCH)], sem)

    def _wait_gather(t, rows, dst_c, sem, sem_c):
        pltpu.make_async_copy(
            dst_hbm.at[pl.ds(wid * EPW + t * CH, CH)], dst_c, sem_c).wait()
        pltpu.make_async_copy(
            xs_hbm.at[src_v.at[pl.ds(t * CH, HCH)]],
            rows.at[pl.ds(0, HCH)], sem).wait()
        pltpu.make_async_copy(
            xs_hbm.at[src_v.at[pl.ds(t * CH + HCH, HCH)]],
            rows.at[pl.ds(HCH, HCH)], sem).wait()

    _gather(0, rows_a, dst_ca, sem_a, sem_ca)

    def _pair(p, _):
        ta = 2 * p
        tb = 2 * p + 1
        _gather(tb, rows_b, dst_cb, sem_b, sem_cb)

        _wait_gather(ta, rows_a, dst_ca, sem_a, sem_ca)
        _scale(ta, rows_a)
        _scatter(rows_a, dst_ca)

        @pl.when(p < NCHA // 2 - 1)
        def _():
            _gather(ta + 2, rows_a, dst_ca, sem_a, sem_ca)

        @pl.when(p == NCHA // 2 - 1)
        def _():
            pltpu.async_copy(
                xs_hbm.at[src_v.at[pl.ds(NCHA * CH, TAIL)]],
                rows_a.at[pl.ds(0, TAIL)], sem_a)

        _wait_gather(tb, rows_b, dst_cb, sem_b, sem_cb)
        _scale(tb, rows_b)
        _scatter(rows_b, dst_cb)
        return 0
    lax.fori_loop(0, NCHA // 2, _pair, 0)

    tk = NCHA * CH
    pltpu.make_async_copy(
        xs_hbm.at[src_v.at[pl.ds(tk, TAIL)]],
        rows_a.at[pl.ds(0, TAIL)], sem_a).wait()
    w16 = nrm_v[pl.ds(tk, 16)]
    for r in range(16):
        spl = w16.at[lax.broadcast(r, (16,))].get(mode="promise_in_bounds")
        for f in range(D // 16):
            rows_a[r, pl.ds(f * 16, 16)] = rows_a[r, pl.ds(f * 16, 16)] * spl
    pltpu.async_copy(rows_a.at[pl.ds(0, TAIL)], agg_sh.at[dst_t],
                     sem_s, add=True).wait()

    plsc.subcore_barrier()
    pltpu.sync_copy(agg_sh.at[pl.ds(sid * SLA, SLA)],
                    out_hbm.at[cid, pl.ds(sid * SLA, SLA)])


def _mm_pre_body(x_ref, w_ref, b_ref, o_ref):
    acc = jnp.dot(x_ref[...], w_ref[...], preferred_element_type=jnp.float32)
    o_ref[...] = jnp.maximum(acc + b_ref[...], 0.0)


def _mm_mid_body(p_ref, w_ref, b_ref, o_ref):
    s = p_ref[0] + p_ref[1]
    acc = jnp.dot(s, w_ref[...], preferred_element_type=jnp.float32)
    o_ref[...] = jnp.maximum(acc + b_ref[...], 0.0)


def _mm_fin_body(p_ref, w_ref, b_ref, wp_ref, bp_ref, o_ref):
    s = p_ref[0] + p_ref[1]
    acc = jnp.dot(s, w_ref[...], preferred_element_type=jnp.float32)
    x = jnp.maximum(acc + b_ref[...], 0.0)
    o_ref[...] = jnp.dot(x, wp_ref[...],
                         preferred_element_type=jnp.float32) + bp_ref[...]


_w_spec = pl.BlockSpec((D, D), lambda i: (0, 0))
_b_spec = pl.BlockSpec((1, D), lambda i: (0, 0))
_row_spec = pl.BlockSpec((RB, D), lambda i: (i, 0))
_p_spec = pl.BlockSpec((NC, RB, D), lambda i: (0, i, 0))
_out_rows = jax.ShapeDtypeStruct((NP, D), jnp.float32)


def _tc_pre(x, w, b):
    return pl.pallas_call(
        _mm_pre_body, grid=(NP // RB,),
        in_specs=[_row_spec, _w_spec, _b_spec],
        out_specs=_row_spec, out_shape=_out_rows,
    )(x, w, b)


def _tc_mid(p, w, b):
    return pl.pallas_call(
        _mm_mid_body, grid=(NP // RB,),
        in_specs=[_p_spec, _w_spec, _b_spec],
        out_specs=_row_spec, out_shape=_out_rows,
    )(p, w, b)


def _tc_fin(p, w, b, wp, bp):
    return pl.pallas_call(
        _mm_fin_body, grid=(NP // RB,),
        in_specs=[_p_spec, _w_spec, _b_spec, _w_spec, _b_spec],
        out_specs=_row_spec, out_shape=_out_rows,
    )(p, w, b, wp, bp)


def kernel(h, edge_index, edge_weight, W_pre, b_pre, W1, b1, W2, b2,
           W_post, b_post):
    src = edge_index[0].astype(jnp.int32)
    dst = edge_index[1].astype(jnp.int32)
    w = edge_weight.astype(jnp.float32)

    norm = _norm_kernel(src, dst, w)

    h_pad = jnp.pad(h, ((0, NP - N), (0, 0)))
    b_pre2 = b_pre.reshape(1, D)
    b12 = b1.reshape(1, D)
    b22 = b2.reshape(1, D)
    b_post2 = b_post.reshape(1, D)

    x1 = _tc_pre(h_pad, W_pre, b_pre2)
    p1 = _agg_kernel(x1, src, dst, norm)
    x2 = _tc_mid(p1, W1, b12)
    p2 = _agg_kernel(x2, src, dst, norm)
    out = _tc_fin(p2, W2, b22, W_post, b_post2)
    return out[:N]


# speedup vs baseline: 2.1657x; 1.0021x over previous
import functools

import jax
import jax.numpy as jnp
from jax import lax
from jax.experimental import pallas as pl
from jax.experimental.pallas import tpu as pltpu
from jax.experimental.pallas import tpu_sc as plsc

N = 10000
E = 320000
D = 128
NC = 2
NS = 16
NW = NC * NS
NP = 10240
EPW = E // NW
EPT = E // NS
CH = 96
NCHA = EPW // CH
TAIL = EPW - NCHA * CH
SL = NP // NS
NA = 10112
SLA = NA // NS
RB = 512
DFD = 25

_mesh = plsc.VectorSubcoreMesh(core_axis_name="c", subcore_axis_name="s")
_sc_params = pltpu.CompilerParams(needs_layout_passes=False)


def _zero_vec16():
    return jnp.zeros((16,), jnp.float32)


@functools.partial(
    pl.kernel,
    out_type=jax.ShapeDtypeStruct((E,), jnp.float32),
    mesh=_mesh,
    scratch_types=[
        pltpu.VMEM((EPW,), jnp.int32),
        pltpu.VMEM((EPT,), jnp.int32),
        pltpu.VMEM((EPT,), jnp.float32),
        pltpu.VMEM((EPW,), jnp.float32),
        pltpu.VMEM((NP,), jnp.float32),
        pltpu.VMEM((SL,), jnp.float32),
        pltpu.VMEM_SHARED((NP,), jnp.float32),
        pltpu.VMEM_SHARED((NP,), jnp.float32),
        pltpu.SemaphoreType.DMA,
        pltpu.SemaphoreType.DMA,
    ],
    compiler_params=_sc_params,
)
def _norm_kernel(src_hbm, dst_hbm, w_hbm, norm_hbm,
                 src_v, dst_v, w_v, nrm_v, dsq_v, sl_v, deg_sh, dsq_sh,
                 sem0, sem1):
    cid = lax.axis_index("c")
    sid = lax.axis_index("s")
    wid = sid * NC + cid

    c_s = pltpu.async_copy(src_hbm.at[pl.ds(wid * EPW, EPW)], src_v, sem0)
    c_d = pltpu.async_copy(dst_hbm.at[pl.ds(sid * EPT, EPT)], dst_v, sem0)
    c_w = pltpu.async_copy(w_hbm.at[pl.ds(sid * EPT, EPT)], w_v, sem0)

    def _z(k, _):
        sl_v[pl.ds(k * 16, 16)] = _zero_vec16()
        return 0
    lax.fori_loop(0, SL // 16, _z, 0)
    pltpu.sync_copy(sl_v, deg_sh.at[pl.ds(sid * SL, SL)])
    c_s.wait()
    c_d.wait()
    c_w.wait()
    plsc.subcore_barrier()

    def _dbatch(t, _):
        descs = []
        for j in range(DFD):
            k = (t * DFD + j) * 16
            idx16 = dst_v[pl.ds(k, 16)]
            descs.append(pltpu.async_copy(
                w_v.at[pl.ds(k, 16)], deg_sh.at[idx16], sem1, add=True))
        for d in descs:
            d.wait()
        return 0
    lax.fori_loop(0, EPT // 16 // DFD, _dbatch, 0)
    plsc.subcore_barrier()

    pltpu.sync_copy(deg_sh.at[pl.ds(sid * SL, SL)], sl_v)

    def _rs(k, _):
        x = jnp.maximum(sl_v[pl.ds(k * 16, 16)], 1e-12)
        i = lax.bitcast_convert_type(x, jnp.int32)
        i = 0x5F3759DF - lax.shift_right_logical(i, 1)
        y = lax.bitcast_convert_type(i, jnp.float32)
        for _ in range(3):
            y = y * (1.5 - 0.5 * x * y * y)
        sl_v[pl.ds(k * 16, 16)] = y
        return 0
    lax.fori_loop(0, SL // 16, _rs, 0)
    pltpu.sync_copy(sl_v, dsq_sh.at[pl.ds(sid * SL, SL)])
    plsc.subcore_barrier()

    pltpu.sync_copy(dsq_sh, dsq_v)
    off = cid * EPW

    def _ngrp(t, _):
        k = t * 16
        s16 = src_v[pl.ds(k, 16)]
        d16 = dst_v[pl.ds(off + k, 16)]
        ww = w_v[pl.ds(off + k, 16)]
        a = plsc.load_gather(dsq_v, [s16])
        b = plsc.load_gather(dsq_v, [d16])
        nrm_v[pl.ds(k, 16)] = ww * a * b
        return 0
    lax.fori_loop(0, EPW // 16, _ngrp, 0)
    pltpu.sync_copy(nrm_v, norm_hbm.at[pl.ds(wid * EPW, EPW)])


@functools.partial(
    pl.kernel,
    out_type=jax.ShapeDtypeStruct((NC, NA, D), jnp.float32),
    mesh=_mesh,
    scratch_types=[
        pltpu.VMEM((EPW,), jnp.int32),
        pltpu.VMEM((16,), jnp.int32),
        pltpu.VMEM((EPW,), jnp.float32),
        pltpu.VMEM((CH, D), jnp.float32),
        pltpu.VMEM((CH, D), jnp.float32),
        pltpu.VMEM((CH,), jnp.int32),
        pltpu.VMEM((CH,), jnp.int32),
        pltpu.VMEM_SHARED((NA, D), jnp.float32),
        pltpu.SemaphoreType.DMA,
        pltpu.SemaphoreType.DMA,
        pltpu.SemaphoreType.DMA,
        pltpu.SemaphoreType.DMA,
        pltpu.SemaphoreType.DMA,
        pltpu.SemaphoreType.DMA,
    ],
    compiler_params=_sc_params,
)
def _agg_kernel(xs_hbm, src_hbm, dst_hbm, nrm_hbm, out_hbm,
                src_v, dst_t, nrm_v, rows_a, rows_b, dst_ca, dst_cb, agg_sh,
                sem0, sem_a, sem_b, sem_s, sem_ca, sem_cb):
    cid = lax.axis_index("c")
    sid = lax.axis_index("s")
    wid = sid * NC + cid

    c_s = pltpu.async_copy(src_hbm.at[pl.ds(wid * EPW, EPW)], src_v, sem0)
    c_d = pltpu.async_copy(
        dst_hbm.at[pl.ds(wid * EPW + NCHA * CH, TAIL)], dst_t, sem0)
    c_n = pltpu.async_copy(nrm_hbm.at[pl.ds(wid * EPW, EPW)], nrm_v, sem0)

    def _z(k, _):
        rows_a[k // 8, pl.ds((k % 8) * 16, 16)] = _zero_vec16()
        return 0
    lax.fori_loop(0, CH * D // 16, _z, 0)
    for k in range(SLA // CH):
        pltpu.sync_copy(rows_a, agg_sh.at[pl.ds(sid * SLA + k * CH, CH)])
    rem = SLA % CH
    pltpu.sync_copy(rows_a.at[pl.ds(0, rem)],
                    agg_sh.at[pl.ds(sid * SLA + SLA - rem, rem)])
    c_s.wait()
    c_d.wait()
    c_n.wait()
    plsc.subcore_barrier()

    def _scale(t, rows):
        def _grp(g, _):
            k = t * CH + g * 16
            w16 = nrm_v[pl.ds(k, 16)]
            base = g * 16
            for r in range(16):
                spl = w16.at[lax.broadcast(r, (16,))].get(
                    mode="promise_in_bounds")
                for f in range(D // 16):
                    rows[base + r, pl.ds(f * 16, 16)] = (
                        rows[base + r, pl.ds(f * 16, 16)] * spl)
            return 0
        lax.fori_loop(0, CH // 16, _grp, 0)

    def _scatter(rows, dst_c):
        pltpu.async_copy(rows, agg_sh.at[dst_c], sem_s, add=True)
        pltpu.make_async_copy(rows, agg_sh.at[dst_c], sem_s).wait()

    HCH = CH // 2

    def _gather(t, rows, dst_c, sem, sem_c):
        pltpu.async_copy(
            dst_hbm.at[pl.ds(wid * EPW + t * CH, CH)], dst_c, sem_c)
        pltpu.async_copy(
            xs_hbm.at[src_v.at[pl.ds(t * CH, HCH)]],
            rows.at[pl.ds(0, HCH)], sem)
        pltpu.async_copy(
            xs_hbm.at[src_v.at[pl.ds(t * CH + HCH, HCH)]],
            rows.at[pl.ds(HCH, HCH)], sem)

    def _wait_gather(t, rows, dst_c, sem, sem_c):
        pltpu.make_async_copy(
            dst_hbm.at[pl.ds(wid * EPW + t * CH, CH)], dst_c, sem_c).wait()
        pltpu.make_async_copy(
            xs_hbm.at[src_v.at[pl.ds(t * CH, HCH)]],
            rows.at[pl.ds(0, HCH)], sem).wait()
        pltpu.make_async_copy(
            xs_hbm.at[src_v.at[pl.ds(t * CH + HCH, HCH)]],
            rows.at[pl.ds(HCH, HCH)], sem).wait()

    _gather(0, rows_a, dst_ca, sem_a, sem_ca)

    def _pair(p, _):
        ta = 2 * p
        tb = 2 * p + 1
        _gather(tb, rows_b, dst_cb, sem_b, sem_cb)

        _wait_gather(ta, rows_a, dst_ca, sem_a, sem_ca)
        _scale(ta, rows_a)
        _scatter(rows_a, dst_ca)

        @pl.when(p < NCHA // 2 - 1)
        def _():
            _gather(ta + 2, rows_a, dst_ca, sem_a, sem_ca)

        @pl.when(p == NCHA // 2 - 1)
        def _():
            pltpu.async_copy(
                xs_hbm.at[src_v.at[pl.ds(NCHA * CH, TAIL)]],
                rows_a.at[pl.ds(0, TAIL)], sem_a)

        _wait_gather(tb, rows_b, dst_cb, sem_b, sem_cb)
        _scale(tb, rows_b)
        _scatter(rows_b, dst_cb)
        return 0
    lax.fori_loop(0, NCHA // 2, _pair, 0)

    tk = NCHA * CH
    pltpu.make_async_copy(
        xs_hbm.at[src_v.at[pl.ds(tk, TAIL)]],
        rows_a.at[pl.ds(0, TAIL)], sem_a).wait()
    w16 = nrm_v[pl.ds(tk, 16)]
    for r in range(16):
        spl = w16.at[lax.broadcast(r, (16,))].get(mode="promise_in_bounds")
        for f in range(D // 16):
            rows_a[r, pl.ds(f * 16, 16)] = rows_a[r, pl.ds(f * 16, 16)] * spl
    pltpu.async_copy(rows_a.at[pl.ds(0, TAIL)], agg_sh.at[dst_t],
                     sem_s, add=True).wait()

    plsc.subcore_barrier()
    pltpu.sync_copy(agg_sh.at[pl.ds(sid * SLA, SLA)],
                    out_hbm.at[cid, pl.ds(sid * SLA, SLA)])


def _mm_pre_body(x_ref, w_ref, b_ref, o_ref):
    acc = jnp.dot(x_ref[...], w_ref[...], preferred_element_type=jnp.float32)
    o_ref[...] = jnp.maximum(acc + b_ref[...], 0.0)


def _mm_mid_body(p_ref, w_ref, b_ref, o_ref):
    s = p_ref[0] + p_ref[1]
    acc = jnp.dot(s, w_ref[...], preferred_element_type=jnp.float32)
    o_ref[...] = jnp.maximum(acc + b_ref[...], 0.0)


def _mm_fin_body(p_ref, w_ref, b_ref, wp_ref, bp_ref, o_ref):
    s = p_ref[0] + p_ref[1]
    acc = jnp.dot(s, w_ref[...], preferred_element_type=jnp.float32)
    x = jnp.maximum(acc + b_ref[...], 0.0)
    o_ref[...] = jnp.dot(x, wp_ref[...],
                         preferred_element_type=jnp.float32) + bp_ref[...]


_w_spec = pl.BlockSpec((D, D), lambda i: (0, 0))
_b_spec = pl.BlockSpec((1, D), lambda i: (0, 0))
_row_spec = pl.BlockSpec((RB, D), lambda i: (i, 0))
_p_spec = pl.BlockSpec((NC, RB, D), lambda i: (0, i, 0))
_out_rows = jax.ShapeDtypeStruct((NP, D), jnp.float32)


def _tc_pre(x, w, b):
    return pl.pallas_call(
        _mm_pre_body, grid=(NP // RB,),
        in_specs=[_row_spec, _w_spec, _b_spec],
        out_specs=_row_spec, out_shape=_out_rows,
    )(x, w, b)


def _tc_mid(p, w, b):
    return pl.pallas_call(
        _mm_mid_body, grid=(NP // RB,),
        in_specs=[_p_spec, _w_spec, _b_spec],
        out_specs=_row_spec, out_shape=_out_rows,
    )(p, w, b)


def _tc_fin(p, w, b, wp, bp):
    return pl.pallas_call(
        _mm_fin_body, grid=(NP // RB,),
        in_specs=[_p_spec, _w_spec, _b_spec, _w_spec, _b_spec],
        out_specs=_row_spec, out_shape=_out_rows,
    )(p, w, b, wp, bp)


def kernel(h, edge_index, edge_weight, W_pre, b_pre, W1, b1, W2, b2,
           W_post, b_post):
    src = edge_index[0].astype(jnp.int32)
    dst = edge_index[1].astype(jnp.int32)
    w = edge_weight.astype(jnp.float32)

    norm = _norm_kernel(src, dst, w)

    h_pad = jnp.pad(h, ((0, NP - N), (0, 0)))
    b_pre2 = b_pre.reshape(1, D)
    b12 = b1.reshape(1, D)
    b22 = b2.reshape(1, D)
    b_post2 = b_post.reshape(1, D)

    x1 = _tc_pre(h_pad, W_pre, b_pre2)
    p1 = _agg_kernel(x1, src, dst, norm)
    x2 = _tc_mid(p1, W1, b12)
    p2 = _agg_kernel(x2, src, dst, norm)
    out = _tc_fin(p2, W2, b22, W_post, b_post2)
    return out[:N]


# speedup vs baseline: 2.1958x; 1.0139x over previous
import functools

import jax
import jax.numpy as jnp
from jax import lax
from jax.experimental import pallas as pl
from jax.experimental.pallas import tpu as pltpu
from jax.experimental.pallas import tpu_sc as plsc

N = 10000
E = 320000
D = 128
NC = 2
NS = 16
NW = NC * NS
NP = 10240
EPW = E // NW
EPT = E // NS
CH = 96
NCHA = EPW // CH
TAIL = EPW - NCHA * CH
SL = NP // NS
NA = 10112
SLA = NA // NS
RB = 512
DFD = 25

_mesh = plsc.VectorSubcoreMesh(core_axis_name="c", subcore_axis_name="s")
_sc_params = pltpu.CompilerParams(needs_layout_passes=False)


def _zero_vec16():
    return jnp.zeros((16,), jnp.float32)


@functools.partial(
    pl.kernel,
    out_type=jax.ShapeDtypeStruct((E,), jnp.float32),
    mesh=_mesh,
    scratch_types=[
        pltpu.VMEM((EPW,), jnp.int32),
        pltpu.VMEM((EPT,), jnp.int32),
        pltpu.VMEM((EPT,), jnp.float32),
        pltpu.VMEM((EPW,), jnp.float32),
        pltpu.VMEM((NP,), jnp.float32),
        pltpu.VMEM((SL,), jnp.float32),
        pltpu.VMEM_SHARED((NP,), jnp.float32),
        pltpu.VMEM_SHARED((NP,), jnp.float32),
        pltpu.SemaphoreType.DMA,
        pltpu.SemaphoreType.DMA,
    ],
    compiler_params=_sc_params,
)
def _norm_kernel(src_hbm, dst_hbm, w_hbm, norm_hbm,
                 src_v, dst_v, w_v, nrm_v, dsq_v, sl_v, deg_sh, dsq_sh,
                 sem0, sem1):
    cid = lax.axis_index("c")
    sid = lax.axis_index("s")
    wid = sid * NC + cid

    c_s = pltpu.async_copy(src_hbm.at[pl.ds(wid * EPW, EPW)], src_v, sem0)
    c_d = pltpu.async_copy(dst_hbm.at[pl.ds(sid * EPT, EPT)], dst_v, sem0)
    c_w = pltpu.async_copy(w_hbm.at[pl.ds(sid * EPT, EPT)], w_v, sem0)

    def _z(k, _):
        sl_v[pl.ds(k * 16, 16)] = _zero_vec16()
        return 0
    lax.fori_loop(0, SL // 16, _z, 0)
    pltpu.sync_copy(sl_v, deg_sh.at[pl.ds(sid * SL, SL)])
    c_s.wait()
    c_d.wait()
    c_w.wait()
    plsc.subcore_barrier()

    def _dbatch(t, _):
        descs = []
        for j in range(DFD):
            k = (t * DFD + j) * 16
            idx16 = dst_v[pl.ds(k, 16)]
            descs.append(pltpu.async_copy(
                w_v.at[pl.ds(k, 16)], deg_sh.at[idx16], sem1, add=True))
        for d in descs:
            d.wait()
        return 0
    lax.fori_loop(0, EPT // 16 // DFD, _dbatch, 0)
    plsc.subcore_barrier()

    pltpu.sync_copy(deg_sh.at[pl.ds(sid * SL, SL)], sl_v)

    def _rs(k, _):
        x = jnp.maximum(sl_v[pl.ds(k * 16, 16)], 1e-12)
        i = lax.bitcast_convert_type(x, jnp.int32)
        i = 0x5F3759DF - lax.shift_right_logical(i, 1)
        y = lax.bitcast_convert_type(i, jnp.float32)
        for _ in range(3):
            y = y * (1.5 - 0.5 * x * y * y)
        sl_v[pl.ds(k * 16, 16)] = y
        return 0
    lax.fori_loop(0, SL // 16, _rs, 0)
    pltpu.sync_copy(sl_v, dsq_sh.at[pl.ds(sid * SL, SL)])
    plsc.subcore_barrier()

    pltpu.sync_copy(dsq_sh, dsq_v)
    off = cid * EPW

    def _ngrp(t, _):
        k = t * 16
        s16 = src_v[pl.ds(k, 16)]
        d16 = dst_v[pl.ds(off + k, 16)]
        ww = w_v[pl.ds(off + k, 16)]
        a = plsc.load_gather(dsq_v, [s16])
        b = plsc.load_gather(dsq_v, [d16])
        nrm_v[pl.ds(k, 16)] = ww * a * b
        return 0
    lax.fori_loop(0, EPW // 16, _ngrp, 0)
    pltpu.sync_copy(nrm_v, norm_hbm.at[pl.ds(wid * EPW, EPW)])


@functools.partial(
    pl.kernel,
    out_type=jax.ShapeDtypeStruct((NC, NA, D), jnp.float32),
    mesh=_mesh,
    scratch_types=[
        pltpu.VMEM((EPW,), jnp.int32),
        pltpu.VMEM((16,), jnp.int32),
        pltpu.VMEM((EPW,), jnp.float32),
        pltpu.VMEM((CH, D), jnp.float32),
        pltpu.VMEM((CH, D), jnp.float32),
        pltpu.VMEM((CH,), jnp.int32),
        pltpu.VMEM((CH,), jnp.int32),
        pltpu.VMEM_SHARED((NA, D), jnp.float32),
        pltpu.SemaphoreType.DMA,
        pltpu.SemaphoreType.DMA,
        pltpu.SemaphoreType.DMA,
        pltpu.SemaphoreType.DMA,
        pltpu.SemaphoreType.DMA,
        pltpu.SemaphoreType.DMA,
    ],
    compiler_params=_sc_params,
)
def _agg_kernel(xs_hbm, src_hbm, dst_hbm, nrm_hbm, out_hbm,
                src_v, dst_t, nrm_v, rows_a, rows_b, dst_ca, dst_cb, agg_sh,
                sem0, sem_a, sem_b, sem_s, sem_ca, sem_cb):
    cid = lax.axis_index("c")
    sid = lax.axis_index("s")
    wid = sid * NC + cid

    c_s = pltpu.async_copy(src_hbm.at[pl.ds(wid * EPW, EPW)], src_v, sem0)
    c_d = pltpu.async_copy(
        dst_hbm.at[pl.ds(wid * EPW + NCHA * CH, TAIL)], dst_t, sem0)
    c_n = pltpu.async_copy(nrm_hbm.at[pl.ds(wid * EPW, EPW)], nrm_v, sem0)

    def _z(k, _):
        rows_a[k // 8, pl.ds((k % 8) * 16, 16)] = _zero_vec16()
        return 0
    lax.fori_loop(0, CH * D // 16, _z, 0)
    for k in range(SLA // CH):
        pltpu.sync_copy(rows_a, agg_sh.at[pl.ds(sid * SLA + k * CH, CH)])
    rem = SLA % CH
    pltpu.sync_copy(rows_a.at[pl.ds(0, rem)],
                    agg_sh.at[pl.ds(sid * SLA + SLA - rem, rem)])
    c_s.wait()
    c_d.wait()
    c_n.wait()
    plsc.subcore_barrier()

    def _scale(t, rows):
        def _grp(g, _):
            k = t * CH + g * 16
            w16 = nrm_v[pl.ds(k, 16)]
            base = g * 16
            for r in range(16):
                spl = w16.at[lax.broadcast(r, (16,))].get(
                    mode="promise_in_bounds")
                for f in range(D // 16):
                    rows[base + r, pl.ds(f * 16, 16)] = (
                        rows[base + r, pl.ds(f * 16, 16)] * spl)
            return 0
        lax.fori_loop(0, CH // 16, _grp, 0)

    def _scatter(rows, dst_c):
        pltpu.async_copy(rows, agg_sh.at[dst_c], sem_s, add=True)
        pltpu.make_async_copy(rows, agg_sh.at[dst_c], sem_s).wait()

    HCH = CH // 2

    def _gather(t, rows, dst_c, sem, sem_c):
        pltpu.async_copy(
            dst_hbm.at[pl.ds(wid * EPW + t * CH, CH)], dst_c, sem_c)
        pltpu.async_copy(
            xs_hbm.at[src_v.at[pl.ds(t * CH, HCH)]],
            rows.at[pl.ds(0, HCH)], sem)
        pltpu.async_copy(
            xs_hbm.at[src_v.at[pl.ds(t * CH + HCH, HCH)]],
            rows.at[pl.ds(HCH, HCH)], sem)

    def _wait_gather(t, rows, dst_c, sem, sem_c):
        pltpu.make_async_copy(
            dst_hbm.at[pl.ds(wid * EPW + t * CH, CH)], dst_c, sem_c).wait()
        pltpu.make_async_copy(
            xs_hbm.at[src_v.at[pl.ds(t * CH, HCH)]],
            rows.at[pl.ds(0, HCH)], sem).wait()
        pltpu.make_async_copy(
            xs_hbm.at[src_v.at[pl.ds(t * CH + HCH, HCH)]],
            rows.at[pl.ds(HCH, HCH)], sem).wait()

    _gather(0, rows_a, dst_ca, sem_a, sem_ca)

    def _pair(p, _):
        ta = 2 * p
        tb = 2 * p + 1
        _gather(tb, rows_b, dst_cb, sem_b, sem_cb)

        _wait_gather(ta, rows_a, dst_ca, sem_a, sem_ca)
        _scale(ta, rows_a)
        _scatter(rows_a, dst_ca)

        @pl.when(p < NCHA // 2 - 1)
        def _():
            _gather(ta + 2, rows_a, dst_ca, sem_a, sem_ca)

        @pl.when(p == NCHA // 2 - 1)
        def _():
            pltpu.async_copy(
                xs_hbm.at[src_v.at[pl.ds(NCHA * CH, TAIL)]],
                rows_a.at[pl.ds(0, TAIL)], sem_a)

        _wait_gather(tb, rows_b, dst_cb, sem_b, sem_cb)
        _scale(tb, rows_b)
        _scatter(rows_b, dst_cb)
        return 0
    lax.fori_loop(0, NCHA // 2, _pair, 0)

    tk = NCHA * CH
    pltpu.make_async_copy(
        xs_hbm.at[src_v.at[pl.ds(tk, TAIL)]],
        rows_a.at[pl.ds(0, TAIL)], sem_a).wait()
    w16 = nrm_v[pl.ds(tk, 16)]
    for r in range(16):
        spl = w16.at[lax.broadcast(r, (16,))].get(mode="promise_in_bounds")
        for f in range(D // 16):
            rows_a[r, pl.ds(f * 16, 16)] = rows_a[r, pl.ds(f * 16, 16)] * spl
    pltpu.async_copy(rows_a.at[pl.ds(0, TAIL)], agg_sh.at[dst_t],
                     sem_s, add=True).wait()

    plsc.subcore_barrier()
    pltpu.sync_copy(agg_sh.at[pl.ds(sid * SLA, SLA)],
                    out_hbm.at[cid, pl.ds(sid * SLA, SLA)])


def _mm_pre_body(x_ref, w_ref, b_ref, o_ref):
    acc = jnp.dot(x_ref[...], w_ref[...], preferred_element_type=jnp.float32)
    o_ref[...] = jnp.maximum(acc + b_ref[...], 0.0)


def _mm_mid_body(p_ref, w_ref, b_ref, o_ref):
    s = p_ref[0] + p_ref[1]
    acc = jnp.dot(s, w_ref[...], preferred_element_type=jnp.float32)
    o_ref[...] = jnp.maximum(acc + b_ref[...], 0.0)


def _mm_fin_body(p_ref, w_ref, b_ref, wp_ref, bp_ref, o_ref):
    s = p_ref[0] + p_ref[1]
    acc = jnp.dot(s, w_ref[...], preferred_element_type=jnp.float32)
    x = jnp.maximum(acc + b_ref[...], 0.0)
    o_ref[...] = jnp.dot(x, wp_ref[...],
                         preferred_element_type=jnp.float32) + bp_ref[...]


_w_spec = pl.BlockSpec((D, D), lambda i: (0, 0))
_b_spec = pl.BlockSpec((1, D), lambda i: (0, 0))
_row_spec = pl.BlockSpec((RB, D), lambda i: (i, 0))
_p_spec = pl.BlockSpec((NC, RB, D), lambda i: (0, i, 0))
_out_rows = jax.ShapeDtypeStruct((NP, D), jnp.float32)


def _tc_pre(x, w, b):
    return pl.pallas_call(
        _mm_pre_body, grid=(NP // RB,),
        in_specs=[_row_spec, _w_spec, _b_spec],
        out_specs=_row_spec, out_shape=_out_rows,
    )(x, w, b)


def _tc_mid(p, w, b):
    return pl.pallas_call(
        _mm_mid_body, grid=(NP // RB,),
        in_specs=[_p_spec, _w_spec, _b_spec],
        out_specs=_row_spec, out_shape=_out_rows,
    )(p, w, b)


def _tc_fin(p, w, b, wp, bp):
    return pl.pallas_call(
        _mm_fin_body, grid=(NP // RB,),
        in_specs=[_p_spec, _w_spec, _b_spec, _w_spec, _b_spec],
        out_specs=_row_spec,
        out_shape=jax.ShapeDtypeStruct((N, D), jnp.float32),
    )(p, w, b, wp, bp)


def kernel(h, edge_index, edge_weight, W_pre, b_pre, W1, b1, W2, b2,
           W_post, b_post):
    src = edge_index[0].astype(jnp.int32)
    dst = edge_index[1].astype(jnp.int32)
    w = edge_weight.astype(jnp.float32)

    norm = _norm_kernel(src, dst, w)

    b_pre2 = b_pre.reshape(1, D)
    b12 = b1.reshape(1, D)
    b22 = b2.reshape(1, D)
    b_post2 = b_post.reshape(1, D)

    x1 = _tc_pre(h, W_pre, b_pre2)
    p1 = _agg_kernel(x1, src, dst, norm)
    x2 = _tc_mid(p1, W1, b12)
    p2 = _agg_kernel(x2, src, dst, norm)
    return _tc_fin(p2, W2, b22, W_post, b_post2)
